# baseline (speedup 1.0000x reference)
import jax
import jax.numpy as jnp
from jax import lax
from jax.experimental import pallas as pl
from jax.experimental.pallas import tpu as pltpu
from jax.experimental.pallas import tpu_sc as plsc

B = 16384
F = 100
D = 32
NC = 2
NS = 16
NW = NC * NS
CB = 512
TBU = CB // 128
NU = F * (B // CB)
PER_W = NU // NW
L = 16


def _sc_body(x_hbm, emb_hbm, bias_hbm, offs_hbm, out_hbm,
             offv, biasv, idx, rows, obuf,
             gsem0, gsem1, osem0, osem1, xsem0, xsem1):
    wid = lax.axis_index("s") * NC + lax.axis_index("c")
    u0 = wid * PER_W

    pltpu.sync_copy(offs_hbm, offv)
    pltpu.sync_copy(bias_hbm, biasv)

    iota = lax.broadcasted_iota(jnp.int32, (L,), 0)
    gsems = (gsem0, gsem1)
    osems = (osem0, osem1)
    xsems = (xsem0, xsem1)

    def unit_fcb(u):
        f = jnp.right_shift(u, 5)
        cb = jnp.bitwise_and(u, 31)
        return f, cb

    def fire_xload(u, buf):
        f, cb = unit_fcb(u)
        src = x_hbm.at[pl.ds(f * B + cb * CB, CB)]
        pltpu.async_copy(src, idx.at[buf], xsems[buf])

    def add_offset(u, buf):
        f, _ = unit_fcb(u)
        off = jnp.full((L,), offv[pl.ds(f, L)][0], jnp.int32)
        ib = idx.at[buf]
        for k in range(CB // L):
            sl = pl.ds(k * L, L)
            ib[sl] = ib[sl] + off

    def fire_gather(buf):
        pltpu.async_copy(emb_hbm.at[idx.at[buf]], rows.at[buf], gsems[buf])

    def wait_gather(buf):
        pltpu.make_async_copy(emb_hbm.at[pl.ds(0, CB)], rows.at[buf],
                              gsems[buf]).wait()

    def fire_out(u, buf):
        f, cb = unit_fcb(u)
        for td in range(D // 8):
            pltpu.async_copy(obuf.at[buf, td, :, :, pl.ds(0, 128)],
                             out_hbm.at[f, td, pl.ds(cb * TBU, TBU)],
                             osems[buf])

    def wait_out(buf):
        for td in range(D // 8):
            pltpu.make_async_copy(obuf.at[buf, td, :, :, pl.ds(0, 128)],
                                  out_hbm.at[0, 0, pl.ds(0, TBU)],
                                  osems[buf]).wait()

    i_r = jnp.bitwise_and(iota, 7)
    i_td = (jnp.right_shift(iota, 3), jnp.right_shift(iota, 3) + 2)

    def transpose_bias(u, buf):
        f, _ = unit_fcb(u)
        rb = rows.at[buf]
        ob = obuf.at[buf]
        bias_h = (biasv[pl.ds(f * D, L)], biasv[pl.ds(f * D + L, L)])

        for tb in range(TBU):
            tbs = jnp.full((L,), tb, jnp.int32)

            @plsc.parallel_loop(0, 32, carry=jnp.zeros((L,), jnp.int32),
                                unroll=4)
            def per_c4(t, cs):
                for dc in range(4):
                    b = tb * 128 + t * 4 + dc
                    v0 = rb[b, pl.ds(0, L)] + bias_h[0]
                    plsc.store_scatter(ob, [i_td[0], tbs, i_r, cs], v0)
                    v1 = rb[b, pl.ds(L, L)] + bias_h[1]
                    plsc.store_scatter(ob, [i_td[1], tbs, i_r, cs], v1)
                    cs = cs + 1
                return cs

    for p in range(2):
        fire_xload(u0 + p, p)
    for p in range(2):
        pltpu.make_async_copy(x_hbm.at[pl.ds(0, CB)], idx.at[p],
                              xsems[p]).wait()
        add_offset(u0 + p, p)
        fire_gather(p)

    def pair_iter(g, _):
        for buf in range(2):
            i = 2 * g + buf
            u = u0 + i
            wait_gather(buf)

            @pl.when(i + 2 < PER_W)
            def _prefetch():
                fire_xload(u + 2, buf)

            @pl.when(i >= 2)
            def _drain():
                wait_out(buf)

            transpose_bias(u, buf)
            fire_out(u, buf)

            @pl.when(i + 2 < PER_W)
            def _next():
                pltpu.make_async_copy(x_hbm.at[pl.ds(0, CB)], idx.at[buf],
                                      xsems[buf]).wait()
                add_offset(u + 2, buf)
                fire_gather(buf)
        return _
    lax.fori_loop(0, PER_W // 2, pair_iter, None)

    for buf in range(2):
        wait_out(buf)


@jax.jit
def kernel(x, emb_weight, bias, category_offsets):
    x_fm = x.T.reshape(B * F)
    bias_flat = jnp.pad(bias.reshape(F * D), (0, 128))
    offs_pad = jnp.pad(category_offsets, (0, 128 - F))

    mesh = plsc.VectorSubcoreMesh(core_axis_name="c", subcore_axis_name="s",
                                  num_cores=NC, num_subcores=NS)
    call = pl.kernel(
        _sc_body,
        out_type=jax.ShapeDtypeStruct((F, D // 8, B // 128, 8, 128),
                                      jnp.float32),
        mesh=mesh,
        compiler_params=pltpu.CompilerParams(use_tc_tiling_on_sc=False,
                                             needs_layout_passes=False),
        scratch_types=[
            pltpu.VMEM((128,), jnp.int32),
            pltpu.VMEM((F * D + 128,), jnp.float32),
            pltpu.VMEM((2, CB), jnp.int32),
            pltpu.VMEM((2, CB, D), jnp.float32),
            pltpu.VMEM((2, D // 8, TBU, 8, 129), jnp.float32),
            pltpu.SemaphoreType.DMA,
            pltpu.SemaphoreType.DMA,
            pltpu.SemaphoreType.DMA,
            pltpu.SemaphoreType.DMA,
            pltpu.SemaphoreType.DMA,
            pltpu.SemaphoreType.DMA,
        ],
    )
    out5 = call(x_fm, emb_weight, bias_flat, offs_pad)
    return out5.transpose(2, 4, 0, 1, 3).reshape(B, F, D)


# speedup vs baseline: 1.0630x; 1.0630x over previous
import jax
import jax.numpy as jnp
from jax import lax
from jax.experimental import pallas as pl
from jax.experimental.pallas import tpu as pltpu
from jax.experimental.pallas import tpu_sc as plsc

B = 16384
F = 100
D = 32
NC = 2
NS = 16
NW = NC * NS
CB = 512
TBU = CB // 128
NU = F * (B // CB)
PER_W = NU // NW
L = 16

V = 1000000
NT = 7872
VP = NT * 128
BLK_W = NT // NW


def _transpose_body(e5_hbm, embL_hbm, tin, tout, isem0, isem1, osem0, osem1):
    wid = lax.axis_index("s") * NC + lax.axis_index("c")
    iota = lax.broadcasted_iota(jnp.int32, (L,), 0)
    isems = (isem0, isem1)
    osems = (osem0, osem1)
    civ = [iota + k * L for k in range(128 // L)]

    def fire_in(i, buf):
        t = wid + i * NW
        pltpu.async_copy(e5_hbm.at[:, t], tin.at[buf], isems[buf])

    def wait_in(buf):
        pltpu.make_async_copy(e5_hbm.at[:, 0], tin.at[buf], isems[buf]).wait()

    def fire_out(i, buf):
        t = wid + i * NW
        pltpu.async_copy(tout.at[buf, :, pl.ds(0, D)],
                         embL_hbm.at[pl.ds(t * 128, 128)], osems[buf])

    def wait_out(buf):
        pltpu.make_async_copy(tout.at[buf, :, pl.ds(0, D)],
                              embL_hbm.at[pl.ds(0, 128)], osems[buf]).wait()

    for p in range(2):
        fire_in(p, p)

    def blk_iter(g, _):
        for buf in range(2):
            i = 2 * g + buf
            wait_in(buf)

            @pl.when(i >= 2)
            def _drain():
                wait_out(buf)

            tb_ = tin.at[buf]
            to_ = tout.at[buf]
            for a in range(4):
                for b in range(8):
                    dsplat = jnp.full((L,), 8 * a + b, jnp.int32)
                    for k in range(128 // L):
                        vec = tb_[a, b, pl.ds(k * L, L)]
                        plsc.store_scatter(to_, [civ[k], dsplat], vec)
            fire_out(i, buf)

            @pl.when(i + 2 < BLK_W)
            def _next():
                fire_in(i + 2, buf)
        return _
    lax.fori_loop(0, BLK_W // 2, blk_iter, None)
    wait_out(0)
    wait_out(1)


def _sc_body(x_hbm, emb_hbm, bias_hbm, offs_hbm, out_hbm,
             offv, biasv, idx, rows, obuf,
             gsem0, gsem1, osem0, osem1, xsem0, xsem1):
    wid = lax.axis_index("s") * NC + lax.axis_index("c")
    u0 = wid * PER_W

    pltpu.sync_copy(offs_hbm, offv)
    pltpu.sync_copy(bias_hbm, biasv)

    iota = lax.broadcasted_iota(jnp.int32, (L,), 0)
    gsems = (gsem0, gsem1)
    osems = (osem0, osem1)
    xsems = (xsem0, xsem1)

    def unit_fcb(u):
        f = jnp.right_shift(u, 5)
        cb = jnp.bitwise_and(u, 31)
        return f, cb

    def fire_xload(u, buf):
        f, cb = unit_fcb(u)
        src = x_hbm.at[pl.ds(f * B + cb * CB, CB)]
        pltpu.async_copy(src, idx.at[buf], xsems[buf])

    def add_offset(u, buf):
        f, _ = unit_fcb(u)
        off = jnp.full((L,), offv[pl.ds(f, L)][0], jnp.int32)
        ib = idx.at[buf]
        for k in range(CB // L):
            sl = pl.ds(k * L, L)
            ib[sl] = ib[sl] + off

    def fire_gather(buf):
        pltpu.async_copy(emb_hbm.at[idx.at[buf]], rows.at[buf], gsems[buf])

    def wait_gather(buf):
        pltpu.make_async_copy(emb_hbm.at[pl.ds(0, CB)], rows.at[buf],
                              gsems[buf]).wait()

    def fire_out(u, buf):
        f, cb = unit_fcb(u)
        for td in range(D // 8):
            pltpu.async_copy(obuf.at[buf, td, :, :, pl.ds(0, 128)],
                             out_hbm.at[f, td, pl.ds(cb * TBU, TBU)],
                             osems[buf])

    def wait_out(buf):
        for td in range(D // 8):
            pltpu.make_async_copy(obuf.at[buf, td, :, :, pl.ds(0, 128)],
                                  out_hbm.at[0, 0, pl.ds(0, TBU)],
                                  osems[buf]).wait()

    i_r = jnp.bitwise_and(iota, 7)
    i_td = (jnp.right_shift(iota, 3), jnp.right_shift(iota, 3) + 2)

    def transpose_bias(u, buf):
        f, _ = unit_fcb(u)
        rb = rows.at[buf]
        ob = obuf.at[buf]
        bias_h = (biasv[pl.ds(f * D, L)], biasv[pl.ds(f * D + L, L)])

        for tb in range(TBU):
            tbs = jnp.full((L,), tb, jnp.int32)

            @plsc.parallel_loop(0, 32, carry=jnp.zeros((L,), jnp.int32),
                                unroll=2)
            def per_c4(t, cs):
                for dc in range(4):
                    b = tb * 128 + t * 4 + dc
                    v0 = rb[b, pl.ds(0, L)] + bias_h[0]
                    plsc.store_scatter(ob, [i_td[0], tbs, i_r, cs], v0)
                    v1 = rb[b, pl.ds(L, L)] + bias_h[1]
                    plsc.store_scatter(ob, [i_td[1], tbs, i_r, cs], v1)
                    cs = cs + 1
                return cs

    for p in range(2):
        fire_xload(u0 + p, p)
    for p in range(2):
        pltpu.make_async_copy(x_hbm.at[pl.ds(0, CB)], idx.at[p],
                              xsems[p]).wait()
        add_offset(u0 + p, p)
        fire_gather(p)

    def pair_iter(g, _):
        for buf in range(2):
            i = 2 * g + buf
            u = u0 + i
            wait_gather(buf)

            @pl.when(i + 2 < PER_W)
            def _prefetch():
                fire_xload(u + 2, buf)

            @pl.when(i >= 2)
            def _drain():
                wait_out(buf)

            transpose_bias(u, buf)
            fire_out(u, buf)

            @pl.when(i + 2 < PER_W)
            def _next():
                pltpu.make_async_copy(x_hbm.at[pl.ds(0, CB)], idx.at[buf],
                                      xsems[buf]).wait()
                add_offset(u + 2, buf)
                fire_gather(buf)
        return _
    lax.fori_loop(0, PER_W // 2, pair_iter, None)

    for buf in range(2):
        wait_out(buf)


@jax.jit
def kernel(x, emb_weight, bias, category_offsets):
    x_fm = x.T.reshape(B * F)
    bias_flat = jnp.pad(bias.reshape(F * D), (0, 128))
    offs_pad = jnp.pad(category_offsets, (0, 128 - F))

    mesh = plsc.VectorSubcoreMesh(core_axis_name="c", subcore_axis_name="s",
                                  num_cores=NC, num_subcores=NS)

    emb5 = (jnp.pad(emb_weight.T, ((0, 0), (0, VP - V)))
            .reshape(4, 8, NT, 128).transpose(0, 2, 1, 3))
    tcall = pl.kernel(
        _transpose_body,
        out_type=jax.ShapeDtypeStruct((VP, D), jnp.float32),
        mesh=mesh,
        compiler_params=pltpu.CompilerParams(use_tc_tiling_on_sc=False,
                                             needs_layout_passes=False),
        scratch_types=[
            pltpu.VMEM((2, 4, 8, 128), jnp.float32),
            pltpu.VMEM((2, 128, 33), jnp.float32),
            pltpu.SemaphoreType.DMA,
            pltpu.SemaphoreType.DMA,
            pltpu.SemaphoreType.DMA,
            pltpu.SemaphoreType.DMA,
        ],
    )
    embL = tcall(emb5)

    call = pl.kernel(
        _sc_body,
        out_type=jax.ShapeDtypeStruct((F, D // 8, B // 128, 8, 128),
                                      jnp.float32),
        mesh=mesh,
        compiler_params=pltpu.CompilerParams(use_tc_tiling_on_sc=False,
                                             needs_layout_passes=False),
        scratch_types=[
            pltpu.VMEM((128,), jnp.int32),
            pltpu.VMEM((F * D + 128,), jnp.float32),
            pltpu.VMEM((2, CB), jnp.int32),
            pltpu.VMEM((2, CB, D), jnp.float32),
            pltpu.VMEM((2, D // 8, TBU, 8, 129), jnp.float32),
            pltpu.SemaphoreType.DMA,
            pltpu.SemaphoreType.DMA,
            pltpu.SemaphoreType.DMA,
            pltpu.SemaphoreType.DMA,
            pltpu.SemaphoreType.DMA,
            pltpu.SemaphoreType.DMA,
        ],
    )
    out5 = call(x_fm, embL, bias_flat, offs_pad)
    return out5.transpose(2, 4, 0, 1, 3).reshape(B, F, D)


# speedup vs baseline: 1.4357x; 1.3506x over previous
import jax
import jax.numpy as jnp
from jax import lax
from jax.experimental import pallas as pl
from jax.experimental.pallas import tpu as pltpu
from jax.experimental.pallas import tpu_sc as plsc

B = 16384
F = 100
D = 32
NC = 2
NS = 16
NW = NC * NS
CB = 512
TBU = CB // 128
NU = F * (B // CB)
PER_W = NU // NW
L = 16

V = 1000000
NT = 7872
VP = NT * 128
BLK_W = NT // NW


def _transpose_body(e5_hbm, embL_hbm, tin, tout, isem0, isem1, osem0, osem1):
    wid = lax.axis_index("s") * NC + lax.axis_index("c")
    iota = lax.broadcasted_iota(jnp.int32, (L,), 0)
    isems = (isem0, isem1)
    osems = (osem0, osem1)
    civ = [iota + k * L for k in range(128 // L)]

    def fire_in(i, buf):
        t = wid + i * NW
        pltpu.async_copy(e5_hbm.at[:, t], tin.at[buf], isems[buf])

    def wait_in(buf):
        pltpu.make_async_copy(e5_hbm.at[:, 0], tin.at[buf], isems[buf]).wait()

    def fire_out(i, buf):
        t = wid + i * NW
        pltpu.async_copy(tout.at[buf, :, pl.ds(0, D)],
                         embL_hbm.at[pl.ds(t * 128, 128)], osems[buf])

    def wait_out(buf):
        pltpu.make_async_copy(tout.at[buf, :, pl.ds(0, D)],
                              embL_hbm.at[pl.ds(0, 128)], osems[buf]).wait()

    for p in range(2):
        fire_in(p, p)

    def blk_iter(g, _):
        for buf in range(2):
            i = 2 * g + buf
            wait_in(buf)

            @pl.when(i >= 2)
            def _drain():
                wait_out(buf)

            tb_ = tin.at[buf]
            to_ = tout.at[buf]

            @plsc.parallel_loop(0, D)
            def _rows(j):
                a = jnp.right_shift(j, 3)
                b = jnp.bitwise_and(j, 7)
                dsplat = jnp.full((L,), j, jnp.int32)
                for k in range(128 // L):
                    vec = tb_[a, b, pl.ds(k * L, L)]
                    plsc.store_scatter(to_, [civ[k], dsplat], vec)
            fire_out(i, buf)

            @pl.when(i + 2 < BLK_W)
            def _next():
                fire_in(i + 2, buf)
        return _
    lax.fori_loop(0, BLK_W // 2, blk_iter, None)
    wait_out(0)
    wait_out(1)


def _sc_body(x_hbm, emb_hbm, bias_hbm, offs_hbm, out_hbm,
             offv, biasv, idx, rows, obuf,
             gsem0, gsem1, osem0, osem1, xsem0, xsem1):
    wid = lax.axis_index("s") * NC + lax.axis_index("c")
    u0 = wid * PER_W

    pltpu.sync_copy(offs_hbm, offv)
    pltpu.sync_copy(bias_hbm, biasv)

    iota = lax.broadcasted_iota(jnp.int32, (L,), 0)
    gsems = (gsem0, gsem1)
    osems = (osem0, osem1)
    xsems = (xsem0, xsem1)

    def unit_fcb(u):
        f = jnp.right_shift(u, 5)
        cb = jnp.bitwise_and(u, 31)
        return f, cb

    def fire_xload(u, buf):
        f, cb = unit_fcb(u)
        src = x_hbm.at[pl.ds(f * B + cb * CB, CB)]
        pltpu.async_copy(src, idx.at[buf], xsems[buf])

    def add_offset(u, buf):
        f, _ = unit_fcb(u)
        off = jnp.full((L,), offv[pl.ds(f, L)][0], jnp.int32)
        ib = idx.at[buf]
        for k in range(CB // L):
            sl = pl.ds(k * L, L)
            ib[sl] = ib[sl] + off

    def fire_gather(buf):
        pltpu.async_copy(emb_hbm.at[idx.at[buf]], rows.at[buf], gsems[buf])

    def wait_gather(buf):
        pltpu.make_async_copy(emb_hbm.at[pl.ds(0, CB)], rows.at[buf],
                              gsems[buf]).wait()

    def fire_out(u, buf):
        f, cb = unit_fcb(u)
        for td in range(D // 8):
            pltpu.async_copy(obuf.at[buf, td, :, :, pl.ds(0, 128)],
                             out_hbm.at[f, td, pl.ds(cb * TBU, TBU)],
                             osems[buf])

    def wait_out(buf):
        for td in range(D // 8):
            pltpu.make_async_copy(obuf.at[buf, td, :, :, pl.ds(0, 128)],
                                  out_hbm.at[0, 0, pl.ds(0, TBU)],
                                  osems[buf]).wait()

    i_r = jnp.bitwise_and(iota, 7)
    i_td = (jnp.right_shift(iota, 3), jnp.right_shift(iota, 3) + 2)

    def transpose_bias(u, buf):
        f, _ = unit_fcb(u)
        rb = rows.at[buf]
        ob = obuf.at[buf]
        bias_h = (biasv[pl.ds(f * D, L)], biasv[pl.ds(f * D + L, L)])

        for tb in range(TBU):
            tbs = jnp.full((L,), tb, jnp.int32)

            @plsc.parallel_loop(0, 32, carry=jnp.zeros((L,), jnp.int32),
                                unroll=2)
            def per_c4(t, cs):
                for dc in range(4):
                    b = tb * 128 + t * 4 + dc
                    v0 = rb[b, pl.ds(0, L)] + bias_h[0]
                    plsc.store_scatter(ob, [i_td[0], tbs, i_r, cs], v0)
                    v1 = rb[b, pl.ds(L, L)] + bias_h[1]
                    plsc.store_scatter(ob, [i_td[1], tbs, i_r, cs], v1)
                    cs = cs + 1
                return cs

    for p in range(2):
        fire_xload(u0 + p, p)
    for p in range(2):
        pltpu.make_async_copy(x_hbm.at[pl.ds(0, CB)], idx.at[p],
                              xsems[p]).wait()
        add_offset(u0 + p, p)
        fire_gather(p)

    def pair_iter(g, _):
        for buf in range(2):
            i = 2 * g + buf
            u = u0 + i
            wait_gather(buf)

            @pl.when(i + 2 < PER_W)
            def _prefetch():
                fire_xload(u + 2, buf)

            @pl.when(i >= 2)
            def _drain():
                wait_out(buf)

            transpose_bias(u, buf)
            fire_out(u, buf)

            @pl.when(i + 2 < PER_W)
            def _next():
                pltpu.make_async_copy(x_hbm.at[pl.ds(0, CB)], idx.at[buf],
                                      xsems[buf]).wait()
                add_offset(u + 2, buf)
                fire_gather(buf)
        return _
    lax.fori_loop(0, PER_W // 2, pair_iter, None)

    for buf in range(2):
        wait_out(buf)


@jax.jit
def kernel(x, emb_weight, bias, category_offsets):
    x_fm = x.T.reshape(B * F)
    bias_flat = jnp.pad(bias.reshape(F * D), (0, 128))
    offs_pad = jnp.pad(category_offsets, (0, 128 - F))

    mesh = plsc.VectorSubcoreMesh(core_axis_name="c", subcore_axis_name="s",
                                  num_cores=NC, num_subcores=NS)

    emb5 = (jnp.pad(emb_weight.T, ((0, 0), (0, VP - V)))
            .reshape(4, 8, NT, 128).transpose(0, 2, 1, 3))
    tcall = pl.kernel(
        _transpose_body,
        out_type=jax.ShapeDtypeStruct((VP, D), jnp.float32),
        mesh=mesh,
        compiler_params=pltpu.CompilerParams(use_tc_tiling_on_sc=False,
                                             needs_layout_passes=False),
        scratch_types=[
            pltpu.VMEM((2, 4, 8, 128), jnp.float32),
            pltpu.VMEM((2, 128, 33), jnp.float32),
            pltpu.SemaphoreType.DMA,
            pltpu.SemaphoreType.DMA,
            pltpu.SemaphoreType.DMA,
            pltpu.SemaphoreType.DMA,
        ],
    )
    embL = tcall(emb5)

    call = pl.kernel(
        _sc_body,
        out_type=jax.ShapeDtypeStruct((F, D // 8, B // 128, 8, 128),
                                      jnp.float32),
        mesh=mesh,
        compiler_params=pltpu.CompilerParams(use_tc_tiling_on_sc=False,
                                             needs_layout_passes=False),
        scratch_types=[
            pltpu.VMEM((128,), jnp.int32),
            pltpu.VMEM((F * D + 128,), jnp.float32),
            pltpu.VMEM((2, CB), jnp.int32),
            pltpu.VMEM((2, CB, D), jnp.float32),
            pltpu.VMEM((2, D // 8, TBU, 8, 129), jnp.float32),
            pltpu.SemaphoreType.DMA,
            pltpu.SemaphoreType.DMA,
            pltpu.SemaphoreType.DMA,
            pltpu.SemaphoreType.DMA,
            pltpu.SemaphoreType.DMA,
            pltpu.SemaphoreType.DMA,
        ],
    )
    out5 = call(x_fm, embL, bias_flat, offs_pad)
    return out5.transpose(2, 4, 0, 1, 3).reshape(B, F, D)


# speedup vs baseline: 1.4409x; 1.0036x over previous
import jax
import jax.numpy as jnp
from jax import lax
from jax.experimental import pallas as pl
from jax.experimental.pallas import tpu as pltpu
from jax.experimental.pallas import tpu_sc as plsc

B = 16384
F = 100
D = 32
NC = 2
NS = 16
NW = NC * NS
CB = 512
TBU = CB // 128
NU = F * (B // CB)
PER_W = NU // NW
L = 16

V = 1000000
NT = 7872
VP = NT * 128
BLK_W = NT // NW


def _transpose_body(e5_hbm, embL_hbm, tin, tout, isem0, isem1, osem0, osem1):
    wid = lax.axis_index("s") * NC + lax.axis_index("c")
    iota = lax.broadcasted_iota(jnp.int32, (L,), 0)
    isems = (isem0, isem1)
    osems = (osem0, osem1)
    civ = [iota + k * L for k in range(128 // L)]

    def fire_in(i, buf):
        t = wid + i * NW
        for a in range(4):
            pltpu.async_copy(e5_hbm.at[a, t], tin.at[buf, pl.ds(a * 8, 8)],
                             isems[buf])

    def wait_in(buf):
        for a in range(4):
            pltpu.make_async_copy(e5_hbm.at[0, 0],
                                  tin.at[buf, pl.ds(a * 8, 8)],
                                  isems[buf]).wait()

    def fire_out(i, buf):
        t = wid + i * NW
        pltpu.async_copy(tout.at[buf, :, pl.ds(0, D)],
                         embL_hbm.at[pl.ds(t * 128, 128)], osems[buf])

    def wait_out(buf):
        pltpu.make_async_copy(tout.at[buf, :, pl.ds(0, D)],
                              embL_hbm.at[pl.ds(0, 128)], osems[buf]).wait()

    for p in range(2):
        fire_in(p, p)

    def blk_iter(g, _):
        for buf in range(2):
            i = 2 * g + buf
            wait_in(buf)

            @pl.when(i >= 2)
            def _drain():
                wait_out(buf)

            tb_ = tin.at[buf]
            to_ = tout.at[buf]

            @plsc.parallel_loop(0, D)
            def _rows(j):
                dsplat = jnp.full((L,), j, jnp.int32)
                for k in range(128 // L):
                    vec = tb_[j, pl.ds(k * L, L)]
                    plsc.store_scatter(to_, [civ[k], dsplat], vec)
            fire_out(i, buf)

            @pl.when(i + 2 < BLK_W)
            def _next():
                fire_in(i + 2, buf)
        return _
    lax.fori_loop(0, BLK_W // 2, blk_iter, None)
    wait_out(0)
    wait_out(1)


def _sc_body(x_hbm, emb_hbm, bias_hbm, offs_hbm, out_hbm,
             offv, biasv, idx, rows, obuf,
             gsem0, gsem1, osem0, osem1, xsem0, xsem1):
    wid = lax.axis_index("s") * NC + lax.axis_index("c")
    u0 = wid * PER_W

    pltpu.sync_copy(offs_hbm, offv)
    pltpu.sync_copy(bias_hbm, biasv)

    iota = lax.broadcasted_iota(jnp.int32, (L,), 0)
    gsems = (gsem0, gsem1)
    osems = (osem0, osem1)
    xsems = (xsem0, xsem1)

    def unit_fcb(u):
        f = jnp.right_shift(u, 5)
        cb = jnp.bitwise_and(u, 31)
        return f, cb

    def fire_xload(u, buf):
        f, cb = unit_fcb(u)
        src = x_hbm.at[pl.ds(f * B + cb * CB, CB)]
        pltpu.async_copy(src, idx.at[buf], xsems[buf])

    def add_offset(u, buf):
        f, _ = unit_fcb(u)
        off = jnp.full((L,), offv[pl.ds(f, L)][0], jnp.int32)
        ib = idx.at[buf]
        for k in range(CB // L):
            sl = pl.ds(k * L, L)
            ib[sl] = ib[sl] + off

    def fire_gather(buf):
        pltpu.async_copy(emb_hbm.at[idx.at[buf]], rows.at[buf], gsems[buf])

    def wait_gather(buf):
        pltpu.make_async_copy(emb_hbm.at[pl.ds(0, CB)], rows.at[buf],
                              gsems[buf]).wait()

    def fire_out(u, buf):
        f, cb = unit_fcb(u)
        for td in range(D // 8):
            pltpu.async_copy(obuf.at[buf, td, :, :, pl.ds(0, 128)],
                             out_hbm.at[f, td, pl.ds(cb * TBU, TBU)],
                             osems[buf])

    def wait_out(buf):
        for td in range(D // 8):
            pltpu.make_async_copy(obuf.at[buf, td, :, :, pl.ds(0, 128)],
                                  out_hbm.at[0, 0, pl.ds(0, TBU)],
                                  osems[buf]).wait()

    i_r = jnp.bitwise_and(iota, 7)
    i_td = (jnp.right_shift(iota, 3), jnp.right_shift(iota, 3) + 2)

    def transpose_bias(u, buf):
        f, _ = unit_fcb(u)
        rb = rows.at[buf]
        ob = obuf.at[buf]
        bias_h = (biasv[pl.ds(f * D, L)], biasv[pl.ds(f * D + L, L)])

        for tb in range(TBU):
            tbs = jnp.full((L,), tb, jnp.int32)

            @plsc.parallel_loop(0, 32, carry=jnp.zeros((L,), jnp.int32),
                                unroll=2)
            def per_c4(t, cs):
                for dc in range(4):
                    b = tb * 128 + t * 4 + dc
                    v0 = rb[b, pl.ds(0, L)] + bias_h[0]
                    plsc.store_scatter(ob, [i_td[0], tbs, i_r, cs], v0)
                    v1 = rb[b, pl.ds(L, L)] + bias_h[1]
                    plsc.store_scatter(ob, [i_td[1], tbs, i_r, cs], v1)
                    cs = cs + 1
                return cs

    for p in range(2):
        fire_xload(u0 + p, p)
    for p in range(2):
        pltpu.make_async_copy(x_hbm.at[pl.ds(0, CB)], idx.at[p],
                              xsems[p]).wait()
        add_offset(u0 + p, p)
        fire_gather(p)

    def pair_iter(g, _):
        for buf in range(2):
            i = 2 * g + buf
            u = u0 + i
            wait_gather(buf)

            @pl.when(i + 2 < PER_W)
            def _prefetch():
                fire_xload(u + 2, buf)

            @pl.when(i >= 2)
            def _drain():
                wait_out(buf)

            transpose_bias(u, buf)
            fire_out(u, buf)

            @pl.when(i + 2 < PER_W)
            def _next():
                pltpu.make_async_copy(x_hbm.at[pl.ds(0, CB)], idx.at[buf],
                                      xsems[buf]).wait()
                add_offset(u + 2, buf)
                fire_gather(buf)
        return _
    lax.fori_loop(0, PER_W // 2, pair_iter, None)

    for buf in range(2):
        wait_out(buf)


@jax.jit
def kernel(x, emb_weight, bias, category_offsets):
    x_fm = x.T.reshape(B * F)
    bias_flat = jnp.pad(bias.reshape(F * D), (0, 128))
    offs_pad = jnp.pad(category_offsets, (0, 128 - F))

    mesh = plsc.VectorSubcoreMesh(core_axis_name="c", subcore_axis_name="s",
                                  num_cores=NC, num_subcores=NS)

    emb5 = (jnp.pad(emb_weight.T, ((0, 0), (0, VP - V)))
            .reshape(4, 8, NT, 128).transpose(0, 2, 1, 3))
    tcall = pl.kernel(
        _transpose_body,
        out_type=jax.ShapeDtypeStruct((VP, D), jnp.float32),
        mesh=mesh,
        compiler_params=pltpu.CompilerParams(use_tc_tiling_on_sc=False,
                                             needs_layout_passes=False),
        scratch_types=[
            pltpu.VMEM((2, 32, 128), jnp.float32),
            pltpu.VMEM((2, 128, 33), jnp.float32),
            pltpu.SemaphoreType.DMA,
            pltpu.SemaphoreType.DMA,
            pltpu.SemaphoreType.DMA,
            pltpu.SemaphoreType.DMA,
        ],
    )
    embL = tcall(emb5)

    call = pl.kernel(
        _sc_body,
        out_type=jax.ShapeDtypeStruct((F, D // 8, B // 128, 8, 128),
                                      jnp.float32),
        mesh=mesh,
        compiler_params=pltpu.CompilerParams(use_tc_tiling_on_sc=False,
                                             needs_layout_passes=False),
        scratch_types=[
            pltpu.VMEM((128,), jnp.int32),
            pltpu.VMEM((F * D + 128,), jnp.float32),
            pltpu.VMEM((2, CB), jnp.int32),
            pltpu.VMEM((2, CB, D), jnp.float32),
            pltpu.VMEM((2, D // 8, TBU, 8, 129), jnp.float32),
            pltpu.SemaphoreType.DMA,
            pltpu.SemaphoreType.DMA,
            pltpu.SemaphoreType.DMA,
            pltpu.SemaphoreType.DMA,
            pltpu.SemaphoreType.DMA,
            pltpu.SemaphoreType.DMA,
        ],
    )
    out5 = call(x_fm, embL, bias_flat, offs_pad)
    return out5.transpose(2, 4, 0, 1, 3).reshape(B, F, D)


# speedup vs baseline: 1.4414x; 1.0004x over previous
import jax
import jax.numpy as jnp
from jax import lax
from jax.experimental import pallas as pl
from jax.experimental.pallas import tpu as pltpu
from jax.experimental.pallas import tpu_sc as plsc

B = 16384
F = 100
D = 32
NC = 2
NS = 16
NW = NC * NS
CB = 512
TBU = CB // 128
NU = F * (B // CB)
PER_W = NU // NW
L = 16

V = 1000000
NT = 7872
VP = NT * 128
BLK_W = NT // NW


def _transpose_body(e5_hbm, embL_hbm, tin, tout, isem0, isem1, osem0, osem1):
    wid = lax.axis_index("s") * NC + lax.axis_index("c")
    iota = lax.broadcasted_iota(jnp.int32, (L,), 0)
    isems = (isem0, isem1)
    osems = (osem0, osem1)
    civ = [iota + k * L for k in range(128 // L)]

    def fire_in(i, buf):
        t = wid + i * NW
        for a in range(4):
            pltpu.async_copy(e5_hbm.at[a, t], tin.at[buf, pl.ds(a * 8, 8)],
                             isems[buf])

    def wait_in(buf):
        for a in range(4):
            pltpu.make_async_copy(e5_hbm.at[0, 0],
                                  tin.at[buf, pl.ds(a * 8, 8)],
                                  isems[buf]).wait()

    def fire_out(i, buf):
        t = wid + i * NW
        pltpu.async_copy(tout.at[buf, :, pl.ds(0, D)],
                         embL_hbm.at[pl.ds(t * 128, 128)], osems[buf])

    def wait_out(buf):
        pltpu.make_async_copy(tout.at[buf, :, pl.ds(0, D)],
                              embL_hbm.at[pl.ds(0, 128)], osems[buf]).wait()

    for p in range(2):
        fire_in(p, p)

    def blk_iter(g, _):
        for buf in range(2):
            i = 2 * g + buf
            wait_in(buf)

            @pl.when(i >= 2)
            def _drain():
                wait_out(buf)

            tb_ = tin.at[buf]
            to_ = tout.at[buf]

            @plsc.parallel_loop(0, D, unroll=2)
            def _rows(j):
                dsplat = jnp.full((L,), j, jnp.int32)
                for k in range(128 // L):
                    vec = tb_[j, pl.ds(k * L, L)]
                    plsc.store_scatter(to_, [civ[k], dsplat], vec)
            fire_out(i, buf)

            @pl.when(i + 2 < BLK_W)
            def _next():
                fire_in(i + 2, buf)
        return _
    lax.fori_loop(0, BLK_W // 2, blk_iter, None)
    wait_out(0)
    wait_out(1)


def _sc_body(x_hbm, emb_hbm, bias_hbm, offs_hbm, out_hbm,
             offv, biasv, idx, rows, obuf,
             gsem0, gsem1, osem0, osem1, xsem0, xsem1):
    wid = lax.axis_index("s") * NC + lax.axis_index("c")
    u0 = wid * PER_W

    pltpu.sync_copy(offs_hbm, offv)
    pltpu.sync_copy(bias_hbm, biasv)

    iota = lax.broadcasted_iota(jnp.int32, (L,), 0)
    gsems = (gsem0, gsem1)
    osems = (osem0, osem1)
    xsems = (xsem0, xsem1)

    def unit_fcb(u):
        f = jnp.right_shift(u, 5)
        cb = jnp.bitwise_and(u, 31)
        return f, cb

    def fire_xload(u, buf):
        f, cb = unit_fcb(u)
        src = x_hbm.at[pl.ds(f * B + cb * CB, CB)]
        pltpu.async_copy(src, idx.at[buf], xsems[buf])

    def add_offset(u, buf):
        f, _ = unit_fcb(u)
        off = jnp.full((L,), offv[pl.ds(f, L)][0], jnp.int32)
        ib = idx.at[buf]
        for k in range(CB // L):
            sl = pl.ds(k * L, L)
            ib[sl] = ib[sl] + off

    def fire_gather(buf):
        pltpu.async_copy(emb_hbm.at[idx.at[buf]], rows.at[buf], gsems[buf])

    def wait_gather(buf):
        pltpu.make_async_copy(emb_hbm.at[pl.ds(0, CB)], rows.at[buf],
                              gsems[buf]).wait()

    def fire_out(u, buf):
        f, cb = unit_fcb(u)
        for td in range(D // 8):
            pltpu.async_copy(obuf.at[buf, td, :, :, pl.ds(0, 128)],
                             out_hbm.at[f, td, pl.ds(cb * TBU, TBU)],
                             osems[buf])

    def wait_out(buf):
        for td in range(D // 8):
            pltpu.make_async_copy(obuf.at[buf, td, :, :, pl.ds(0, 128)],
                                  out_hbm.at[0, 0, pl.ds(0, TBU)],
                                  osems[buf]).wait()

    i_r = jnp.bitwise_and(iota, 7)
    i_td = (jnp.right_shift(iota, 3), jnp.right_shift(iota, 3) + 2)

    def transpose_bias(u, buf):
        f, _ = unit_fcb(u)
        rb = rows.at[buf]
        ob = obuf.at[buf]
        bias_h = (biasv[pl.ds(f * D, L)], biasv[pl.ds(f * D + L, L)])

        for tb in range(TBU):
            tbs = jnp.full((L,), tb, jnp.int32)

            @plsc.parallel_loop(0, 32, carry=jnp.zeros((L,), jnp.int32),
                                unroll=2)
            def per_c4(t, cs):
                for dc in range(4):
                    b = tb * 128 + t * 4 + dc
                    v0 = rb[b, pl.ds(0, L)] + bias_h[0]
                    plsc.store_scatter(ob, [i_td[0], tbs, i_r, cs], v0)
                    v1 = rb[b, pl.ds(L, L)] + bias_h[1]
                    plsc.store_scatter(ob, [i_td[1], tbs, i_r, cs], v1)
                    cs = cs + 1
                return cs

    for p in range(2):
        fire_xload(u0 + p, p)
    for p in range(2):
        pltpu.make_async_copy(x_hbm.at[pl.ds(0, CB)], idx.at[p],
                              xsems[p]).wait()
        add_offset(u0 + p, p)
        fire_gather(p)

    def pair_iter(g, _):
        for buf in range(2):
            i = 2 * g + buf
            u = u0 + i
            wait_gather(buf)

            @pl.when(i + 2 < PER_W)
            def _prefetch():
                fire_xload(u + 2, buf)

            @pl.when(i >= 2)
            def _drain():
                wait_out(buf)

            transpose_bias(u, buf)
            fire_out(u, buf)

            @pl.when(i + 2 < PER_W)
            def _next():
                pltpu.make_async_copy(x_hbm.at[pl.ds(0, CB)], idx.at[buf],
                                      xsems[buf]).wait()
                add_offset(u + 2, buf)
                fire_gather(buf)
        return _
    lax.fori_loop(0, PER_W // 2, pair_iter, None)

    for buf in range(2):
        wait_out(buf)


@jax.jit
def kernel(x, emb_weight, bias, category_offsets):
    x_fm = x.T.reshape(B * F)
    bias_flat = jnp.pad(bias.reshape(F * D), (0, 128))
    offs_pad = jnp.pad(category_offsets, (0, 128 - F))

    mesh = plsc.VectorSubcoreMesh(core_axis_name="c", subcore_axis_name="s",
                                  num_cores=NC, num_subcores=NS)

    emb5 = (jnp.pad(emb_weight.T, ((0, 0), (0, VP - V)))
            .reshape(4, 8, NT, 128).transpose(0, 2, 1, 3))
    tcall = pl.kernel(
        _transpose_body,
        out_type=jax.ShapeDtypeStruct((VP, D), jnp.float32),
        mesh=mesh,
        compiler_params=pltpu.CompilerParams(use_tc_tiling_on_sc=False,
                                             needs_layout_passes=False),
        scratch_types=[
            pltpu.VMEM((2, 32, 128), jnp.float32),
            pltpu.VMEM((2, 128, 33), jnp.float32),
            pltpu.SemaphoreType.DMA,
            pltpu.SemaphoreType.DMA,
            pltpu.SemaphoreType.DMA,
            pltpu.SemaphoreType.DMA,
        ],
    )
    embL = tcall(emb5)

    call = pl.kernel(
        _sc_body,
        out_type=jax.ShapeDtypeStruct((F, D // 8, B // 128, 8, 128),
                                      jnp.float32),
        mesh=mesh,
        compiler_params=pltpu.CompilerParams(use_tc_tiling_on_sc=False,
                                             needs_layout_passes=False),
        scratch_types=[
            pltpu.VMEM((128,), jnp.int32),
            pltpu.VMEM((F * D + 128,), jnp.float32),
            pltpu.VMEM((2, CB), jnp.int32),
            pltpu.VMEM((2, CB, D), jnp.float32),
            pltpu.VMEM((2, D // 8, TBU, 8, 129), jnp.float32),
            pltpu.SemaphoreType.DMA,
            pltpu.SemaphoreType.DMA,
            pltpu.SemaphoreType.DMA,
            pltpu.SemaphoreType.DMA,
            pltpu.SemaphoreType.DMA,
            pltpu.SemaphoreType.DMA,
        ],
    )
    out5 = call(x_fm, embL, bias_flat, offs_pad)
    return out5.transpose(2, 4, 0, 1, 3).reshape(B, F, D)
